# baseline (speedup 1.0000x reference)
import functools

import jax
import jax.numpy as jnp
from jax import lax
from jax.experimental import pallas as pl
from jax.experimental.pallas import tpu as pltpu
from jax.experimental.pallas import tpu_sc as plsc
from jax.experimental.layout import Format, Layout, with_layout_constraint

B = 1024
D = 64
C = 16384
NUMY = 1000000
MAX_LEAF = 100
BEAM = 20
ALPHA = 0.5
S = BEAM * MAX_LEAF


_CBLK = 2048


def _matcher_body(e_ref, w_ref, b_ref, scores_ref):
    e = e_ref[...]
    w = w_ref[...]
    logits = lax.dot_general(
        e, w, (((1,), (1,)), ((), ())),
        preferred_element_type=jnp.float32,
    )
    bias = b_ref[...].reshape(_CBLK)
    scores_ref[...] = jax.nn.sigmoid(logits + bias[None, :])


def _matcher(embs, WC_w, WC_b):
    grid = (C // _CBLK,)
    return pl.pallas_call(
        _matcher_body,
        grid=grid,
        in_specs=[
            pl.BlockSpec((B, D), lambda i: (0, 0)),
            pl.BlockSpec((_CBLK, D), lambda i: (i, 0)),
            pl.BlockSpec((_CBLK // 128, 128), lambda i: (i, 0)),
        ],
        out_specs=pl.BlockSpec((B, _CBLK), lambda i: (0, i)),
        out_shape=jax.ShapeDtypeStruct((B, C), jnp.float32),
    )(embs, WC_w, WC_b.reshape(C // 128, 128))


_NW = 32
_QPW = B // _NW
_CHUNK = 128
_NCHUNK = (S + _CHUNK - 1) // _CHUNK
_NBUF = 6
_DUNROLL = 16


def _sc_body(embs_hbm, slinds_hbm, pa_hbm, bias_hbm, wlw_hbm, out_hbm,
             emb_v, pa_v, bias_v, score_v, stg_v, *rest):
    idxbufs = rest[:_NBUF]
    rbufs = rest[_NBUF:2 * _NBUF]
    sems = rest[2 * _NBUF:]
    wid = lax.axis_index("s") * 2 + lax.axis_index("c")
    iota16 = lax.iota(jnp.int32, 16)
    scat_base = iota16 * 16

    def chunk_rows(c):
        return S - c * _CHUNK if c == _NCHUNK - 1 else _CHUNK

    def fire(c, b):
        n = chunk_rows(c)
        ibuf = idxbufs[c % _NBUF]
        idx = ibuf if n == _CHUNK else ibuf.at[pl.ds(0, n)]
        pltpu.sync_copy(slinds_hbm.at[pl.ds(b * S + c * _CHUNK, n)], idx)
        buf = rbufs[c % _NBUF]
        dst = buf if n == _CHUNK else buf.at[pl.ds(0, n)]
        sem = sems[c % _NBUF]
        return pltpu.async_copy(wlw_hbm.at[idx], dst, sem)

    def per_query(t, carry):
        b = wid * _QPW + t
        pltpu.sync_copy(embs_hbm.at[pl.ds(b * D, D)], emb_v)
        pltpu.sync_copy(pa_hbm.at[pl.ds(b * S, S)], pa_v)
        pltpu.sync_copy(bias_hbm.at[pl.ds(b * S, S)], bias_v)
        e_vecs = [emb_v[pl.ds(k * 16, 16)] for k in range(D // 16)]

        descs = {}
        for c in range(min(_NBUF, _NCHUNK)):
            descs[c] = fire(c, b)

        for c in range(_NCHUNK):
            descs.pop(c).wait()
            n = chunk_rows(c)
            buf = rbufs[c % _NBUF]

            def group_body(g, _, c=c, buf=buf):
                off = c * _CHUNK + g * 16
                for rr in range(16):
                    r = g * 16 + rr
                    p = buf[r, pl.ds(0, 16)] * e_vecs[0]
                    for k in range(1, D // 16):
                        p = p + buf[r, pl.ds(k * 16, 16)] * e_vecs[k]
                    plsc.store_scatter(stg_v, [scat_base + rr], p)
                logit = stg_v[pl.ds(0, 16)]
                for j in range(1, 16):
                    logit = logit + stg_v[pl.ds(j * 16, 16)]
                logit = logit + bias_v[pl.ds(off, 16)]
                sig = 1.0 / (1.0 + jnp.exp(-logit))
                score_v[pl.ds(off, 16)] = (
                    pa_v[pl.ds(off, 16)] * (ALPHA * sig + (1.0 - ALPHA)))
                return 0

            lax.fori_loop(0, n // 16, group_body, 0)

            nxt = c + _NBUF
            if nxt < _NCHUNK:
                descs[nxt] = fire(nxt, b)

        pltpu.sync_copy(score_v, out_hbm.at[pl.ds(b * S, S)])
        return carry

    lax.fori_loop(0, _QPW, per_query, 0)


def _sc_sparse(embs, sl_inds, pa, bias_pre, WL_w):
    wl_lin = with_layout_constraint(
        WL_w, Layout(major_to_minor=(0, 1), tiling=((16,),)))
    mesh = plsc.VectorSubcoreMesh(core_axis_name="c", subcore_axis_name="s")
    scratch = [
        pltpu.VMEM((D,), jnp.float32),
        pltpu.VMEM((S,), jnp.float32),
        pltpu.VMEM((S,), jnp.float32),
        pltpu.VMEM((S,), jnp.float32),
        pltpu.VMEM((256,), jnp.float32),
    ]
    scratch += [pltpu.VMEM((_CHUNK,), jnp.int32) for _ in range(_NBUF)]
    scratch += [pltpu.VMEM((_CHUNK, D), jnp.float32) for _ in range(_NBUF)]
    scratch += [pltpu.SemaphoreType.DMA for _ in range(_NBUF)]
    f = pl.kernel(
        _sc_body,
        out_type=jax.ShapeDtypeStruct((B * S,), jnp.float32),
        mesh=mesh,
        scratch_types=scratch,
        compiler_params=pltpu.CompilerParams(needs_layout_passes=False),
    )
    out = f(embs.reshape(-1), sl_inds.reshape(-1), pa.reshape(-1),
            bias_pre.reshape(-1), wl_lin)
    return out.reshape(B, S)


def kernel(xfts, WC_w, WC_b, WL_w, WL_b, A_nz_inds, A_nz_vals):
    embs = xfts / (jnp.linalg.norm(xfts, axis=-1, keepdims=True) + 1e-12)
    cluster_scores = _matcher(embs, WC_w, WC_b)
    top_vals, top_inds = lax.top_k(cluster_scores, BEAM)
    sl_inds = jnp.take(A_nz_inds, top_inds, axis=0, mode="clip").reshape(B, S)
    sl_avals = jnp.take(A_nz_vals, top_inds, axis=0, mode="clip").reshape(B, S)
    pa = jnp.repeat(top_vals, MAX_LEAF, axis=1) * sl_avals
    bias_pre = jnp.take(WL_b, sl_inds.reshape(-1), mode="clip").reshape(B, S)
    scores = _sc_sparse(embs, sl_inds, pa, bias_pre, WL_w)
    return scores, sl_inds


# speedup vs baseline: 1.0508x; 1.0508x over previous
import functools

import jax
import jax.numpy as jnp
from jax import lax
from jax.experimental import pallas as pl
from jax.experimental.pallas import tpu as pltpu
from jax.experimental.pallas import tpu_sc as plsc
from jax.experimental.layout import Format, Layout, with_layout_constraint

B = 1024
D = 64
C = 16384
NUMY = 1000000
MAX_LEAF = 100
BEAM = 20
ALPHA = 0.5
S = BEAM * MAX_LEAF


_RBLK = 128
_KPAD = 32


def _matcher_body(e_ref, w_ref, b_ref, tv_ref, ti_ref, s_ref):
    e = e_ref[...]
    w = w_ref[...]
    logits = lax.dot_general(
        e, w, (((1,), (1,)), ((), ())),
        preferred_element_type=jnp.float32,
    )
    bias = b_ref[...].reshape(C)
    s_ref[...] = jax.nn.sigmoid(logits + bias[None, :])

    def step(k, carry):
        s = s_ref[...]
        m = jnp.max(s, axis=1, keepdims=True)
        iota = lax.broadcasted_iota(jnp.int32, (_RBLK, C), 1)
        idx = jnp.min(jnp.where(s == m, iota, jnp.int32(2**30)),
                      axis=1, keepdims=True)
        tv_ref[pl.ds(k, 1), :] = m.reshape(1, _RBLK)
        ti_ref[pl.ds(k, 1), :] = idx.reshape(1, _RBLK)
        s_ref[...] = jnp.where(iota == idx, -jnp.inf, s)
        return carry

    lax.fori_loop(0, BEAM, step, 0)


def _matcher_topk(embs, WC_w, WC_b):
    grid = (B // _RBLK,)
    return pl.pallas_call(
        _matcher_body,
        grid=grid,
        in_specs=[
            pl.BlockSpec((_RBLK, D), lambda i: (i, 0)),
            pl.BlockSpec((C, D), lambda i: (0, 0)),
            pl.BlockSpec((C // 128, 128), lambda i: (0, 0)),
        ],
        out_specs=[
            pl.BlockSpec((_KPAD, _RBLK), lambda i: (0, i)),
            pl.BlockSpec((_KPAD, _RBLK), lambda i: (0, i)),
        ],
        out_shape=[
            jax.ShapeDtypeStruct((_KPAD, B), jnp.float32),
            jax.ShapeDtypeStruct((_KPAD, B), jnp.int32),
        ],
        scratch_shapes=[pltpu.VMEM((_RBLK, C), jnp.float32)],
    )(embs, WC_w, WC_b.reshape(C // 128, 128))


_NW = 32
_QPW = B // _NW
_CHUNK = 128
_NCHUNK = (S + _CHUNK - 1) // _CHUNK
_NBUF = 6
_DUNROLL = 16


def _sc_body(embs_hbm, slinds_hbm, pa_hbm, bias_hbm, wlw_hbm, out_hbm,
             emb_v, pa_v, bias_v, score_v, stg_v, *rest):
    idxbufs = rest[:_NBUF]
    rbufs = rest[_NBUF:2 * _NBUF]
    sems = rest[2 * _NBUF:]
    wid = lax.axis_index("s") * 2 + lax.axis_index("c")
    iota16 = lax.iota(jnp.int32, 16)
    scat_base = iota16 * 16

    def chunk_rows(c):
        return S - c * _CHUNK if c == _NCHUNK - 1 else _CHUNK

    def fire(c, b):
        n = chunk_rows(c)
        ibuf = idxbufs[c % _NBUF]
        idx = ibuf if n == _CHUNK else ibuf.at[pl.ds(0, n)]
        pltpu.sync_copy(slinds_hbm.at[pl.ds(b * S + c * _CHUNK, n)], idx)
        buf = rbufs[c % _NBUF]
        dst = buf if n == _CHUNK else buf.at[pl.ds(0, n)]
        sem = sems[c % _NBUF]
        return pltpu.async_copy(wlw_hbm.at[idx], dst, sem)

    def per_query(t, carry):
        b = wid * _QPW + t
        pltpu.sync_copy(embs_hbm.at[pl.ds(b * D, D)], emb_v)
        pltpu.sync_copy(pa_hbm.at[pl.ds(b * S, S)], pa_v)
        pltpu.sync_copy(bias_hbm.at[pl.ds(b * S, S)], bias_v)
        e_vecs = [emb_v[pl.ds(k * 16, 16)] for k in range(D // 16)]

        descs = {}
        for c in range(min(_NBUF, _NCHUNK)):
            descs[c] = fire(c, b)

        for c in range(_NCHUNK):
            descs.pop(c).wait()
            n = chunk_rows(c)
            buf = rbufs[c % _NBUF]

            def group_body(g, _, c=c, buf=buf):
                off = c * _CHUNK + g * 16
                for rr in range(16):
                    r = g * 16 + rr
                    p = buf[r, pl.ds(0, 16)] * e_vecs[0]
                    for k in range(1, D // 16):
                        p = p + buf[r, pl.ds(k * 16, 16)] * e_vecs[k]
                    plsc.store_scatter(stg_v, [scat_base + rr], p)
                logit = stg_v[pl.ds(0, 16)]
                for j in range(1, 16):
                    logit = logit + stg_v[pl.ds(j * 16, 16)]
                logit = logit + bias_v[pl.ds(off, 16)]
                sig = 1.0 / (1.0 + jnp.exp(-logit))
                score_v[pl.ds(off, 16)] = (
                    pa_v[pl.ds(off, 16)] * (ALPHA * sig + (1.0 - ALPHA)))
                return 0

            lax.fori_loop(0, n // 16, group_body, 0)

            nxt = c + _NBUF
            if nxt < _NCHUNK:
                descs[nxt] = fire(nxt, b)

        pltpu.sync_copy(score_v, out_hbm.at[pl.ds(b * S, S)])
        return carry

    lax.fori_loop(0, _QPW, per_query, 0)


def _sc_sparse(embs, sl_inds, pa, bias_pre, WL_w):
    wl_lin = with_layout_constraint(
        WL_w, Layout(major_to_minor=(0, 1), tiling=((16,),)))
    mesh = plsc.VectorSubcoreMesh(core_axis_name="c", subcore_axis_name="s")
    scratch = [
        pltpu.VMEM((D,), jnp.float32),
        pltpu.VMEM((S,), jnp.float32),
        pltpu.VMEM((S,), jnp.float32),
        pltpu.VMEM((S,), jnp.float32),
        pltpu.VMEM((256,), jnp.float32),
    ]
    scratch += [pltpu.VMEM((_CHUNK,), jnp.int32) for _ in range(_NBUF)]
    scratch += [pltpu.VMEM((_CHUNK, D), jnp.float32) for _ in range(_NBUF)]
    scratch += [pltpu.SemaphoreType.DMA for _ in range(_NBUF)]
    f = pl.kernel(
        _sc_body,
        out_type=jax.ShapeDtypeStruct((B * S,), jnp.float32),
        mesh=mesh,
        scratch_types=scratch,
        compiler_params=pltpu.CompilerParams(needs_layout_passes=False),
    )
    out = f(embs.reshape(-1), sl_inds.reshape(-1), pa.reshape(-1),
            bias_pre.reshape(-1), wl_lin)
    return out.reshape(B, S)


def kernel(xfts, WC_w, WC_b, WL_w, WL_b, A_nz_inds, A_nz_vals):
    embs = xfts / (jnp.linalg.norm(xfts, axis=-1, keepdims=True) + 1e-12)
    tv_pad, ti_pad = _matcher_topk(embs, WC_w, WC_b)
    top_vals, top_inds = tv_pad[:BEAM].T, ti_pad[:BEAM].T
    sl_inds = jnp.take(A_nz_inds, top_inds, axis=0, mode="clip").reshape(B, S)
    sl_avals = jnp.take(A_nz_vals, top_inds, axis=0, mode="clip").reshape(B, S)
    pa = jnp.repeat(top_vals, MAX_LEAF, axis=1) * sl_avals
    bias_pre = jnp.take(WL_b, sl_inds.reshape(-1), mode="clip").reshape(B, S)
    scores = _sc_sparse(embs, sl_inds, pa, bias_pre, WL_w)
    return scores, sl_inds
